# Initial kernel scaffold; baseline (speedup 1.0000x reference)
import jax
import jax.numpy as jnp
from jax.experimental import pallas as pl


def kernel(x, edge_attr, edge_index, batch, W1a, b1a, W1b, b1b, root1, bias1, W2a, b2a, W2b, b2b, root2, bias2, W3a, b3a, W3b, b3b, root3, bias3, fW1, fb1, fW2, fb2, fW3, fb3):
    raise NotImplementedError("write your pallas kernel here")


# speedup vs baseline: 2.2063x; 2.2063x over previous
import functools

import jax
import jax.numpy as jnp
from jax.experimental import pallas as pl
from jax.experimental.pallas import tpu as pltpu

NEG = 0.01
BF = jnp.bfloat16
F32 = jnp.float32


def _lrelu(v):
    return jnp.maximum(v, NEG * v)


def _split(v):
    hi = v.astype(BF)
    lo = (v - hi.astype(F32)).astype(BF)
    return hi, lo


def _dot_sel(oh_bf, b32, dn=None):
    b_hi, b_lo = _split(b32)
    if dn is None:
        return (jnp.dot(oh_bf, b_hi, preferred_element_type=F32) +
                jnp.dot(oh_bf, b_lo, preferred_element_type=F32))
    return (jax.lax.dot_general(oh_bf, b_hi, dimension_numbers=dn,
                                preferred_element_type=F32) +
            jax.lax.dot_general(oh_bf, b_lo, dimension_numbers=dn,
                                preferred_element_type=F32))


def _dot_x(a32, b32):
    a_hi, a_lo = _split(a32)
    b_hi, b_lo = _split(b32)
    return (jnp.dot(a_hi, b_hi, preferred_element_type=F32) +
            jnp.dot(a_hi, b_lo, preferred_element_type=F32) +
            jnp.dot(a_lo, b_hi, preferred_element_type=F32))


def _layer_body(dT3h_ref, dT3l_ref, dfull_ref, x_ref, ea_ref, wa_ref,
                ba_ref, wb_ref, bbb_ref, root_ref, bias_ref, src_ref,
                dst_ref, out_ref, oh_ref, ohT_ref, h_ref, msg_ref,
                *, E, Nn, K, O, Ib, NI):
    ib = pl.program_id(0)

    @pl.when(ib == 0)
    def _init():
        a = ba_ref[...].astype(F32) * jnp.ones((E, 1), F32)
        for c in range(4):
            a = a + ea_ref[:, c:c + 1] * wa_ref[c:c + 1, :]
        h_ref[...] = _lrelu(a).astype(BF)
        ids = jax.lax.broadcasted_iota(jnp.int32, (E, Nn), 1)
        oh_ref[...] = (src_ref[...] == ids).astype(BF)
        msg_ref[...] = jnp.zeros((E, O), F32)

    dn_t = (((1,), (1,)), ((), ()))
    xs_blk = (jax.lax.dot_general(oh_ref[...], dT3h_ref[0],
                                  dimension_numbers=dn_t,
                                  preferred_element_type=F32) +
              jax.lax.dot_general(oh_ref[...], dT3l_ref[0],
                                  dimension_numbers=dn_t,
                                  preferred_element_type=F32))

    h = h_ref[...]
    acc = msg_ref[...]
    for j in range(Ib):
        wbj = wb_ref[:, j * O:(j + 1) * O].astype(BF)
        wmj = jnp.dot(h, wbj, preferred_element_type=F32)
        wmj = wmj + bbb_ref[j:j + 1, :]
        acc = acc + wmj * xs_blk[:, j:j + 1]
    msg_ref[...] = acc

    @pl.when(ib == NI - 1)
    def _fin():
        idn = jax.lax.broadcasted_iota(jnp.int32, (Nn, E), 0)
        ohT_ref[...] = (dst_ref[...] == idn).astype(BF)
        s = _dot_sel(ohT_ref[...], msg_ref[...])
        cnt = jnp.sum(ohT_ref[...].astype(F32), axis=1, keepdims=True)
        mean = s / jnp.maximum(cnt, 1.0)
        o = mean + _dot_x(dfull_ref[...], root_ref[...]) + bias_ref[...]
        out_ref[:, :O] = _lrelu(o)
        out_ref[:, O:] = x_ref[...]


def _nnconv_layer(d_prev, x, ea, src_col, dst_row, Wa, ba, Wb, bb, root,
                  bias, O, Ib):
    E = ea.shape[0]
    Nn, I = d_prev.shape
    K = Wa.shape[1]
    NI = I // Ib
    assert NI * Ib == I
    Dx = x.shape[1]
    dT = d_prev.T.reshape(NI, Ib, Nn)
    dT_hi = dT.astype(BF)
    dT_lo = (dT - dT_hi.astype(F32)).astype(BF)
    bbb = bb.reshape(I, O)
    body = functools.partial(_layer_body, E=E, Nn=Nn, K=K, O=O, Ib=Ib, NI=NI)
    return pl.pallas_call(
        body,
        grid=(NI,),
        in_specs=[
            pl.BlockSpec((1, Ib, Nn), lambda i: (i, 0, 0)),
            pl.BlockSpec((1, Ib, Nn), lambda i: (i, 0, 0)),
            pl.BlockSpec((Nn, I), lambda i: (0, 0)),
            pl.BlockSpec((Nn, Dx), lambda i: (0, 0)),
            pl.BlockSpec((E, 4), lambda i: (0, 0)),
            pl.BlockSpec((4, K), lambda i: (0, 0)),
            pl.BlockSpec((1, K), lambda i: (0, 0)),
            pl.BlockSpec((K, Ib * O), lambda i: (0, i)),
            pl.BlockSpec((Ib, O), lambda i: (i, 0)),
            pl.BlockSpec((I, O), lambda i: (0, 0)),
            pl.BlockSpec((1, O), lambda i: (0, 0)),
            pl.BlockSpec((E, 1), lambda i: (0, 0)),
            pl.BlockSpec((1, E), lambda i: (0, 0)),
        ],
        out_specs=pl.BlockSpec((Nn, O + Dx), lambda i: (0, 0)),
        out_shape=jax.ShapeDtypeStruct((Nn, O + Dx), jnp.float32),
        scratch_shapes=[
            pltpu.VMEM((E, Nn), BF),
            pltpu.VMEM((Nn, E), BF),
            pltpu.VMEM((E, K), BF),
            pltpu.VMEM((E, O), F32),
        ],
        compiler_params=pltpu.CompilerParams(
            dimension_semantics=("arbitrary",),
            vmem_limit_bytes=100 * 1024 * 1024),
    )(dT_hi, dT_lo, d_prev, x, ea, Wa, ba.reshape(1, K), Wb, bbb, root,
      bias.reshape(1, O), src_col, dst_row)


def _head_body(d_ref, batch_ref, w1_ref, b1_ref, w2_ref, b2_ref, w3_ref,
               b3_ref, out_ref, *, G, Nn):
    idg = jax.lax.broadcasted_iota(jnp.int32, (G, Nn), 0)
    ohB = (batch_ref[...] == idg).astype(BF)
    s = _dot_sel(ohB, d_ref[...])
    cnt = jnp.sum(ohB.astype(F32), axis=1, keepdims=True)
    mean = s / jnp.maximum(cnt, 1.0)
    a1 = _lrelu(_dot_x(mean, w1_ref[...]) + b1_ref[...])
    a2 = _lrelu(_dot_x(a1, w2_ref[...]) + b2_ref[...])
    out_ref[...] = _dot_x(a2, w3_ref[...]) + b3_ref[...]


def _head(d3, batch_row, fW1, fb1, fW2, fb2, fW3, fb3):
    G = 50
    Nn, F = d3.shape
    body = functools.partial(_head_body, G=G, Nn=Nn)
    return pl.pallas_call(
        body,
        out_shape=jax.ShapeDtypeStruct((G, 1), jnp.float32),
    )(d3, batch_row, fW1, fb1.reshape(1, -1), fW2, fb2.reshape(1, -1),
      fW3, fb3.reshape(1, -1))


def kernel(x, edge_attr, edge_index, batch, W1a, b1a, W1b, b1b, root1,
           bias1, W2a, b2a, W2b, b2b, root2, bias2, W3a, b3a, W3b, b3b,
           root3, bias3, fW1, fb1, fW2, fb2, fW3, fb3):
    E = edge_attr.shape[0]
    Nn = x.shape[0]
    src_col = edge_index[0].reshape(E, 1)
    dst_row = edge_index[1].reshape(1, E)
    batch_row = batch.reshape(1, Nn)
    d1 = _nnconv_layer(x, x, edge_attr, src_col, dst_row, W1a, b1a, W1b,
                       b1b, root1, bias1, O=256, Ib=8)
    d2 = _nnconv_layer(d1, x, edge_attr, src_col, dst_row, W2a, b2a, W2b,
                       b2b, root2, bias2, O=256, Ib=8)
    d3 = _nnconv_layer(d2, x, edge_attr, src_col, dst_row, W3a, b3a, W3b,
                       b3b, root3, bias3, O=512, Ib=8)
    return _head(d3, batch_row, fW1, fb1, fW2, fb2, fW3, fb3)


# speedup vs baseline: 2.3003x; 1.0426x over previous
import functools

import jax
import jax.numpy as jnp
from jax.experimental import pallas as pl
from jax.experimental.pallas import tpu as pltpu

NEG = 0.01
BF = jnp.bfloat16
F32 = jnp.float32


def _lrelu(v):
    return jnp.maximum(v, NEG * v)


def _split(v):
    hi = v.astype(BF)
    lo = (v - hi.astype(F32)).astype(BF)
    return hi, lo


def _dot_sel(oh_bf, b32, dn=None):
    b_hi, b_lo = _split(b32)
    if dn is None:
        return (jnp.dot(oh_bf, b_hi, preferred_element_type=F32) +
                jnp.dot(oh_bf, b_lo, preferred_element_type=F32))
    return (jax.lax.dot_general(oh_bf, b_hi, dimension_numbers=dn,
                                preferred_element_type=F32) +
            jax.lax.dot_general(oh_bf, b_lo, dimension_numbers=dn,
                                preferred_element_type=F32))


def _dot_x(a32, b32):
    a_hi, a_lo = _split(a32)
    b_hi, b_lo = _split(b32)
    return (jnp.dot(a_hi, b_hi, preferred_element_type=F32) +
            jnp.dot(a_hi, b_lo, preferred_element_type=F32) +
            jnp.dot(a_lo, b_hi, preferred_element_type=F32))


def _layer_body(dT3hl_ref, dfull_ref, x_ref, ea_ref, wa_ref,
                ba_ref, wb_ref, bbb_ref, root_ref, bias_ref, src_ref,
                dst_ref, out_ref, oh_ref, ohT_ref, h_ref, msg_ref,
                *, E, Nn, K, O, Ib, NI):
    ib = pl.program_id(0)

    @pl.when(ib == 0)
    def _init():
        a = jnp.dot(ea_ref[...].astype(BF), wa_ref[...].astype(BF),
                    preferred_element_type=F32) + ba_ref[...]
        h_ref[...] = _lrelu(a).astype(BF)
        ids = jax.lax.broadcasted_iota(jnp.int32, (E, Nn), 1)
        oh_ref[...] = (src_ref[...] == ids).astype(BF)
        msg_ref[...] = jnp.zeros((E, O), F32)

    dn_t = (((1,), (1,)), ((), ()))
    xs2 = jax.lax.dot_general(oh_ref[...], dT3hl_ref[0],
                              dimension_numbers=dn_t,
                              preferred_element_type=F32)
    xs_blk = xs2[:, :Ib] + xs2[:, Ib:]

    wm = jnp.dot(h_ref[...], wb_ref[...].astype(BF),
                 preferred_element_type=F32)
    acc = msg_ref[...]
    for j in range(Ib):
        wmj = (wm[:, j * O:(j + 1) * O]
               + bbb_ref[j:j + 1, :]).astype(BF).astype(F32)
        xsj = xs_blk[:, j:j + 1].astype(BF).astype(F32)
        acc = acc + wmj * xsj
    msg_ref[...] = acc

    @pl.when(ib == NI - 1)
    def _fin():
        idn = jax.lax.broadcasted_iota(jnp.int32, (Nn, E), 0)
        ohT_ref[...] = (dst_ref[...] == idn).astype(BF)
        s = _dot_sel(ohT_ref[...], msg_ref[...])
        cnt = jnp.sum(ohT_ref[...].astype(F32), axis=1, keepdims=True)
        mean = s / jnp.maximum(cnt, 1.0)
        o = mean + jnp.dot(dfull_ref[...].astype(BF),
                           root_ref[...].astype(BF),
                           preferred_element_type=F32) + bias_ref[...]
        out_ref[:, :O] = _lrelu(o)
        out_ref[:, O:] = x_ref[...]


def _nnconv_layer(d_prev, x, ea, src_col, dst_row, Wa, ba, Wb, bb, root,
                  bias, O, Ib):
    E = ea.shape[0]
    Nn, I = d_prev.shape
    K = Wa.shape[1]
    NI = I // Ib
    assert NI * Ib == I
    Dx = x.shape[1]
    dT = d_prev.T.reshape(NI, Ib, Nn)
    dT_hi = dT.astype(BF)
    dT_lo = (dT - dT_hi.astype(F32)).astype(BF)
    dT_hl = jnp.concatenate([dT_hi, dT_lo], axis=1)
    bbb = bb.reshape(I, O)
    body = functools.partial(_layer_body, E=E, Nn=Nn, K=K, O=O, Ib=Ib, NI=NI)
    return pl.pallas_call(
        body,
        grid=(NI,),
        in_specs=[
            pl.BlockSpec((1, 2 * Ib, Nn), lambda i: (i, 0, 0)),
            pl.BlockSpec((Nn, I), lambda i: (0, 0)),
            pl.BlockSpec((Nn, Dx), lambda i: (0, 0)),
            pl.BlockSpec((E, 4), lambda i: (0, 0)),
            pl.BlockSpec((4, K), lambda i: (0, 0)),
            pl.BlockSpec((1, K), lambda i: (0, 0)),
            pl.BlockSpec((K, Ib * O), lambda i: (0, i)),
            pl.BlockSpec((Ib, O), lambda i: (i, 0)),
            pl.BlockSpec((I, O), lambda i: (0, 0)),
            pl.BlockSpec((1, O), lambda i: (0, 0)),
            pl.BlockSpec((E, 1), lambda i: (0, 0)),
            pl.BlockSpec((1, E), lambda i: (0, 0)),
        ],
        out_specs=pl.BlockSpec((Nn, O + Dx), lambda i: (0, 0)),
        out_shape=jax.ShapeDtypeStruct((Nn, O + Dx), jnp.float32),
        scratch_shapes=[
            pltpu.VMEM((E, Nn), BF),
            pltpu.VMEM((Nn, E), BF),
            pltpu.VMEM((E, K), BF),
            pltpu.VMEM((E, O), F32),
        ],
        compiler_params=pltpu.CompilerParams(
            dimension_semantics=("arbitrary",),
            vmem_limit_bytes=100 * 1024 * 1024),
    )(dT_hl, d_prev, x, ea, Wa, ba.reshape(1, K), Wb, bbb, root,
      bias.reshape(1, O), src_col, dst_row)


def _head_body(d_ref, batch_ref, w1_ref, b1_ref, w2_ref, b2_ref, w3_ref,
               b3_ref, out_ref, *, G, Nn):
    idg = jax.lax.broadcasted_iota(jnp.int32, (G, Nn), 0)
    ohB = (batch_ref[...] == idg).astype(BF)
    s = _dot_sel(ohB, d_ref[...])
    cnt = jnp.sum(ohB.astype(F32), axis=1, keepdims=True)
    mean = s / jnp.maximum(cnt, 1.0)
    a1 = _lrelu(jnp.dot(mean.astype(BF), w1_ref[...].astype(BF),
                        preferred_element_type=F32) + b1_ref[...])
    a2 = _lrelu(jnp.dot(a1.astype(BF), w2_ref[...].astype(BF),
                        preferred_element_type=F32) + b2_ref[...])
    out_ref[...] = jnp.dot(a2.astype(BF), w3_ref[...].astype(BF),
                           preferred_element_type=F32) + b3_ref[...]


def _head(d3, batch_row, fW1, fb1, fW2, fb2, fW3, fb3):
    G = 50
    Nn, F = d3.shape
    body = functools.partial(_head_body, G=G, Nn=Nn)
    return pl.pallas_call(
        body,
        out_shape=jax.ShapeDtypeStruct((G, 1), jnp.float32),
    )(d3, batch_row, fW1, fb1.reshape(1, -1), fW2, fb2.reshape(1, -1),
      fW3, fb3.reshape(1, -1))


def kernel(x, edge_attr, edge_index, batch, W1a, b1a, W1b, b1b, root1,
           bias1, W2a, b2a, W2b, b2b, root2, bias2, W3a, b3a, W3b, b3b,
           root3, bias3, fW1, fb1, fW2, fb2, fW3, fb3):
    E = edge_attr.shape[0]
    Nn = x.shape[0]
    src_col = edge_index[0].reshape(E, 1)
    dst_row = edge_index[1].reshape(1, E)
    batch_row = batch.reshape(1, Nn)
    d1 = _nnconv_layer(x, x, edge_attr, src_col, dst_row, W1a, b1a, W1b,
                       b1b, root1, bias1, O=256, Ib=8)
    d2 = _nnconv_layer(d1, x, edge_attr, src_col, dst_row, W2a, b2a, W2b,
                       b2b, root2, bias2, O=256, Ib=8)
    d3 = _nnconv_layer(d2, x, edge_attr, src_col, dst_row, W3a, b3a, W3b,
                       b3b, root3, bias3, O=512, Ib=8)
    return _head(d3, batch_row, fW1, fb1, fW2, fb2, fW3, fb3)


# speedup vs baseline: 2.3922x; 1.0400x over previous
import functools

import jax
import jax.numpy as jnp
from jax.experimental import pallas as pl
from jax.experimental.pallas import tpu as pltpu

NEG = 0.01
BF = jnp.bfloat16
F32 = jnp.float32


def _lrelu(v):
    return jnp.maximum(v, NEG * v)


def _split(v):
    hi = v.astype(BF)
    lo = (v - hi.astype(F32)).astype(BF)
    return hi, lo


def _dot_sel(oh_bf, b32, dn=None):
    b_hi, b_lo = _split(b32)
    if dn is None:
        return (jnp.dot(oh_bf, b_hi, preferred_element_type=F32) +
                jnp.dot(oh_bf, b_lo, preferred_element_type=F32))
    return (jax.lax.dot_general(oh_bf, b_hi, dimension_numbers=dn,
                                preferred_element_type=F32) +
            jax.lax.dot_general(oh_bf, b_lo, dimension_numbers=dn,
                                preferred_element_type=F32))


def _dot_x(a32, b32):
    a_hi, a_lo = _split(a32)
    b_hi, b_lo = _split(b32)
    return (jnp.dot(a_hi, b_hi, preferred_element_type=F32) +
            jnp.dot(a_hi, b_lo, preferred_element_type=F32) +
            jnp.dot(a_lo, b_hi, preferred_element_type=F32))


def _layer_body(dT3hl_ref, dfull_ref, x_ref, ea_ref, wa_ref,
                ba_ref, wb_ref, bbb_ref, root_ref, bias_ref, src_ref,
                dst_ref, out_ref, oh_ref, ohT_ref, h_ref, msg_ref,
                *, E, Nn, K, O, Ib, NI):
    ib = pl.program_id(0)

    @pl.when(ib == 0)
    def _init():
        a = jnp.dot(ea_ref[...].astype(BF), wa_ref[...].astype(BF),
                    preferred_element_type=F32) + ba_ref[...]
        h_ref[...] = _lrelu(a).astype(BF)
        ids = jax.lax.broadcasted_iota(jnp.int32, (E, Nn), 1)
        oh_ref[...] = (src_ref[...] == ids).astype(BF)
        msg_ref[...] = jnp.zeros((E, O), F32)

    dn_t = (((1,), (1,)), ((), ()))
    xs_blk = jax.lax.dot_general(oh_ref[...], dT3hl_ref[0],
                                 dimension_numbers=dn_t,
                                 preferred_element_type=F32)

    wm = jnp.dot(h_ref[...], wb_ref[...].astype(BF),
                 preferred_element_type=F32)
    acc = msg_ref[...]
    for j in range(Ib):
        wmj = (wm[:, j * O:(j + 1) * O]
               + bbb_ref[j:j + 1, :]).astype(BF).astype(F32)
        acc = acc + wmj * xs_blk[:, j:j + 1]
    msg_ref[...] = acc

    @pl.when(ib == NI - 1)
    def _fin():
        idn = jax.lax.broadcasted_iota(jnp.int32, (Nn, E), 0)
        ohT_ref[...] = (dst_ref[...] == idn).astype(BF)
        s = _dot_sel(ohT_ref[...], msg_ref[...])
        cnt = jnp.sum(ohT_ref[...].astype(F32), axis=1, keepdims=True)
        mean = s / jnp.maximum(cnt, 1.0)
        o = mean + jnp.dot(dfull_ref[...].astype(BF),
                           root_ref[...].astype(BF),
                           preferred_element_type=F32) + bias_ref[...]
        out_ref[:, :O] = _lrelu(o)
        out_ref[:, O:] = x_ref[...]


def _nnconv_layer(d_prev, x, ea, src_col, dst_row, Wa, ba, Wb, bb, root,
                  bias, O, Ib):
    E = ea.shape[0]
    Nn, I = d_prev.shape
    K = Wa.shape[1]
    NI = I // Ib
    assert NI * Ib == I
    Dx = x.shape[1]
    dT_hl = d_prev.T.reshape(NI, Ib, Nn).astype(BF)
    bbb = bb.reshape(I, O)
    body = functools.partial(_layer_body, E=E, Nn=Nn, K=K, O=O, Ib=Ib, NI=NI)
    return pl.pallas_call(
        body,
        grid=(NI,),
        in_specs=[
            pl.BlockSpec((1, Ib, Nn), lambda i: (i, 0, 0)),
            pl.BlockSpec((Nn, I), lambda i: (0, 0)),
            pl.BlockSpec((Nn, Dx), lambda i: (0, 0)),
            pl.BlockSpec((E, 4), lambda i: (0, 0)),
            pl.BlockSpec((4, K), lambda i: (0, 0)),
            pl.BlockSpec((1, K), lambda i: (0, 0)),
            pl.BlockSpec((K, Ib * O), lambda i: (0, i)),
            pl.BlockSpec((Ib, O), lambda i: (i, 0)),
            pl.BlockSpec((I, O), lambda i: (0, 0)),
            pl.BlockSpec((1, O), lambda i: (0, 0)),
            pl.BlockSpec((E, 1), lambda i: (0, 0)),
            pl.BlockSpec((1, E), lambda i: (0, 0)),
        ],
        out_specs=pl.BlockSpec((Nn, O + Dx), lambda i: (0, 0)),
        out_shape=jax.ShapeDtypeStruct((Nn, O + Dx), jnp.float32),
        scratch_shapes=[
            pltpu.VMEM((E, Nn), BF),
            pltpu.VMEM((Nn, E), BF),
            pltpu.VMEM((E, K), BF),
            pltpu.VMEM((E, O), F32),
        ],
        compiler_params=pltpu.CompilerParams(
            dimension_semantics=("arbitrary",),
            vmem_limit_bytes=100 * 1024 * 1024),
    )(dT_hl, d_prev, x, ea, Wa, ba.reshape(1, K), Wb, bbb, root,
      bias.reshape(1, O), src_col, dst_row)


def _head_body(d_ref, batch_ref, w1_ref, b1_ref, w2_ref, b2_ref, w3_ref,
               b3_ref, out_ref, *, G, Nn):
    idg = jax.lax.broadcasted_iota(jnp.int32, (G, Nn), 0)
    ohB = (batch_ref[...] == idg).astype(BF)
    s = _dot_sel(ohB, d_ref[...])
    cnt = jnp.sum(ohB.astype(F32), axis=1, keepdims=True)
    mean = s / jnp.maximum(cnt, 1.0)
    a1 = _lrelu(jnp.dot(mean.astype(BF), w1_ref[...].astype(BF),
                        preferred_element_type=F32) + b1_ref[...])
    a2 = _lrelu(jnp.dot(a1.astype(BF), w2_ref[...].astype(BF),
                        preferred_element_type=F32) + b2_ref[...])
    out_ref[...] = jnp.dot(a2.astype(BF), w3_ref[...].astype(BF),
                           preferred_element_type=F32) + b3_ref[...]


def _head(d3, batch_row, fW1, fb1, fW2, fb2, fW3, fb3):
    G = 50
    Nn, F = d3.shape
    body = functools.partial(_head_body, G=G, Nn=Nn)
    return pl.pallas_call(
        body,
        out_shape=jax.ShapeDtypeStruct((G, 1), jnp.float32),
    )(d3, batch_row, fW1, fb1.reshape(1, -1), fW2, fb2.reshape(1, -1),
      fW3, fb3.reshape(1, -1))


def kernel(x, edge_attr, edge_index, batch, W1a, b1a, W1b, b1b, root1,
           bias1, W2a, b2a, W2b, b2b, root2, bias2, W3a, b3a, W3b, b3b,
           root3, bias3, fW1, fb1, fW2, fb2, fW3, fb3):
    E = edge_attr.shape[0]
    Nn = x.shape[0]
    src_col = edge_index[0].reshape(E, 1)
    dst_row = edge_index[1].reshape(1, E)
    batch_row = batch.reshape(1, Nn)
    d1 = _nnconv_layer(x, x, edge_attr, src_col, dst_row, W1a, b1a, W1b,
                       b1b, root1, bias1, O=256, Ib=8)
    d2 = _nnconv_layer(d1, x, edge_attr, src_col, dst_row, W2a, b2a, W2b,
                       b2b, root2, bias2, O=256, Ib=8)
    d3 = _nnconv_layer(d2, x, edge_attr, src_col, dst_row, W3a, b3a, W3b,
                       b3b, root3, bias3, O=512, Ib=8)
    return _head(d3, batch_row, fW1, fb1, fW2, fb2, fW3, fb3)


# speedup vs baseline: 2.4323x; 1.0167x over previous
import functools

import jax
import jax.numpy as jnp
from jax.experimental import pallas as pl
from jax.experimental.pallas import tpu as pltpu

NEG = 0.01
BF = jnp.bfloat16
F32 = jnp.float32


def _lrelu(v):
    return jnp.maximum(v, NEG * v)


def _split(v):
    hi = v.astype(BF)
    lo = (v - hi.astype(F32)).astype(BF)
    return hi, lo


def _dot_sel(oh_bf, b32, dn=None):
    b_hi, b_lo = _split(b32)
    if dn is None:
        return (jnp.dot(oh_bf, b_hi, preferred_element_type=F32) +
                jnp.dot(oh_bf, b_lo, preferred_element_type=F32))
    return (jax.lax.dot_general(oh_bf, b_hi, dimension_numbers=dn,
                                preferred_element_type=F32) +
            jax.lax.dot_general(oh_bf, b_lo, dimension_numbers=dn,
                                preferred_element_type=F32))


def _dot_x(a32, b32):
    a_hi, a_lo = _split(a32)
    b_hi, b_lo = _split(b32)
    return (jnp.dot(a_hi, b_hi, preferred_element_type=F32) +
            jnp.dot(a_hi, b_lo, preferred_element_type=F32) +
            jnp.dot(a_lo, b_hi, preferred_element_type=F32))


def _layer_body(dT3hl_ref, dfull_ref, x_ref, ea_ref, wa_ref,
                ba_ref, wb_ref, bbb_ref, root_ref, bias_ref, src_ref,
                dst_ref, out_ref, oh_ref, ohT_ref, h_ref, msg_ref,
                *, E, Nn, K, O, Ib, NI):
    ib = pl.program_id(0)

    @pl.when(ib == 0)
    def _init():
        a = jnp.dot(ea_ref[...].astype(BF), wa_ref[...].astype(BF),
                    preferred_element_type=F32) + ba_ref[...]
        h_ref[...] = _lrelu(a).astype(BF)
        ids = jax.lax.broadcasted_iota(jnp.int32, (E, Nn), 1)
        oh_ref[...] = (src_ref[...] == ids).astype(BF)
        msg_ref[...] = jnp.zeros((E, O), F32)

    dn_t = (((1,), (1,)), ((), ()))
    xs_blk = jax.lax.dot_general(oh_ref[...], dT3hl_ref[0],
                                 dimension_numbers=dn_t,
                                 preferred_element_type=F32)

    wm = jnp.dot(h_ref[...], wb_ref[...].astype(BF),
                 preferred_element_type=F32)
    acc = msg_ref[...]
    for j in range(Ib):
        wmj = (wm[:, j * O:(j + 1) * O]
               + bbb_ref[j:j + 1, :]).astype(BF).astype(F32)
        acc = acc + wmj * xs_blk[:, j:j + 1]
    msg_ref[...] = acc

    @pl.when(ib == NI - 1)
    def _fin():
        idn = jax.lax.broadcasted_iota(jnp.int32, (Nn, E), 0)
        ohT_ref[...] = (dst_ref[...] == idn).astype(BF)
        s = _dot_sel(ohT_ref[...], msg_ref[...])
        cnt = jnp.sum(ohT_ref[...].astype(F32), axis=1, keepdims=True)
        mean = s / jnp.maximum(cnt, 1.0)
        o = mean + jnp.dot(dfull_ref[...].astype(BF),
                           root_ref[...].astype(BF),
                           preferred_element_type=F32) + bias_ref[...]
        out_ref[:, :O] = _lrelu(o)
        out_ref[:, O:] = x_ref[...]


def _nnconv_layer(d_prev, x, ea, src_col, dst_row, Wa, ba, Wb, bb, root,
                  bias, O, Ib):
    E = ea.shape[0]
    Nn, I = d_prev.shape
    K = Wa.shape[1]
    NI = I // Ib
    assert NI * Ib == I
    Dx = x.shape[1]
    dT_hl = d_prev.T.reshape(NI, Ib, Nn).astype(BF)
    bbb = bb.reshape(I, O)
    body = functools.partial(_layer_body, E=E, Nn=Nn, K=K, O=O, Ib=Ib, NI=NI)
    return pl.pallas_call(
        body,
        grid=(NI,),
        in_specs=[
            pl.BlockSpec((1, Ib, Nn), lambda i: (i, 0, 0)),
            pl.BlockSpec((Nn, I), lambda i: (0, 0)),
            pl.BlockSpec((Nn, Dx), lambda i: (0, 0)),
            pl.BlockSpec((E, 4), lambda i: (0, 0)),
            pl.BlockSpec((4, K), lambda i: (0, 0)),
            pl.BlockSpec((1, K), lambda i: (0, 0)),
            pl.BlockSpec((K, Ib * O), lambda i: (0, i)),
            pl.BlockSpec((Ib, O), lambda i: (i, 0)),
            pl.BlockSpec((I, O), lambda i: (0, 0)),
            pl.BlockSpec((1, O), lambda i: (0, 0)),
            pl.BlockSpec((E, 1), lambda i: (0, 0)),
            pl.BlockSpec((1, E), lambda i: (0, 0)),
        ],
        out_specs=pl.BlockSpec((Nn, O + Dx), lambda i: (0, 0)),
        out_shape=jax.ShapeDtypeStruct((Nn, O + Dx), jnp.float32),
        scratch_shapes=[
            pltpu.VMEM((E, Nn), BF),
            pltpu.VMEM((Nn, E), BF),
            pltpu.VMEM((E, K), BF),
            pltpu.VMEM((E, O), F32),
        ],
        compiler_params=pltpu.CompilerParams(
            dimension_semantics=("arbitrary",),
            vmem_limit_bytes=100 * 1024 * 1024),
    )(dT_hl, d_prev, x, ea, Wa, ba.reshape(1, K), Wb, bbb, root,
      bias.reshape(1, O), src_col, dst_row)


def _head_body(d_ref, batch_ref, w1_ref, b1_ref, w2_ref, b2_ref, w3_ref,
               b3_ref, out_ref, *, G, Nn):
    idg = jax.lax.broadcasted_iota(jnp.int32, (G, Nn), 0)
    ohB = (batch_ref[...] == idg).astype(BF)
    s = _dot_sel(ohB, d_ref[...])
    cnt = jnp.sum(ohB.astype(F32), axis=1, keepdims=True)
    mean = s / jnp.maximum(cnt, 1.0)
    a1 = _lrelu(jnp.dot(mean.astype(BF), w1_ref[...].astype(BF),
                        preferred_element_type=F32) + b1_ref[...])
    a2 = _lrelu(jnp.dot(a1.astype(BF), w2_ref[...].astype(BF),
                        preferred_element_type=F32) + b2_ref[...])
    out_ref[...] = jnp.dot(a2.astype(BF), w3_ref[...].astype(BF),
                           preferred_element_type=F32) + b3_ref[...]


def _head(d3, batch_row, fW1, fb1, fW2, fb2, fW3, fb3):
    G = 50
    Nn, F = d3.shape
    body = functools.partial(_head_body, G=G, Nn=Nn)
    return pl.pallas_call(
        body,
        out_shape=jax.ShapeDtypeStruct((G, 1), jnp.float32),
    )(d3, batch_row, fW1, fb1.reshape(1, -1), fW2, fb2.reshape(1, -1),
      fW3, fb3.reshape(1, -1))


def kernel(x, edge_attr, edge_index, batch, W1a, b1a, W1b, b1b, root1,
           bias1, W2a, b2a, W2b, b2b, root2, bias2, W3a, b3a, W3b, b3b,
           root3, bias3, fW1, fb1, fW2, fb2, fW3, fb3):
    E = edge_attr.shape[0]
    Nn = x.shape[0]
    src_col = edge_index[0].reshape(E, 1)
    dst_row = edge_index[1].reshape(1, E)
    batch_row = batch.reshape(1, Nn)
    d1 = _nnconv_layer(x, x, edge_attr, src_col, dst_row, W1a, b1a, W1b,
                       b1b, root1, bias1, O=256, Ib=16)
    d2 = _nnconv_layer(d1, x, edge_attr, src_col, dst_row, W2a, b2a, W2b,
                       b2b, root2, bias2, O=256, Ib=16)
    d3 = _nnconv_layer(d2, x, edge_attr, src_col, dst_row, W3a, b3a, W3b,
                       b3b, root3, bias3, O=512, Ib=8)
    return _head(d3, batch_row, fW1, fb1, fW2, fb2, fW3, fb3)
